# baseline (speedup 1.0000x reference)
import functools

import jax
import jax.numpy as jnp
from jax import lax
from jax.experimental import pallas as pl
from jax.experimental.pallas import tpu as pltpu
from jax.experimental.pallas import tpu_sc as plsc

N_NODES = 10000
N_EDGES = 320000
D = 128

NC = 2
NS = 16
CHUNK = 64

NT = NC * NS
IBLK = 8
NRING = 4
PAD_UNIT = NT * CHUNK * IBLK * 2
EDGES_PAD = ((N_EDGES + PAD_UNIT - 1) // PAD_UNIT) * PAD_UNIT
EDGES_PER_TILE = EDGES_PAD // NT
CPT = EDGES_PER_TILE // CHUNK
BLOCKS = CPT // IBLK

ACC_ROWS = 10240
ZERO_PER_TILE = ACC_ROWS // NS
WRITE_PER_TILE = (N_NODES // NS) // 8 * 8
WRITE_TAIL = N_NODES - NS * WRITE_PER_TILE


def _linear_body(feat_ref, w_ref, b_ref, out_ref):
    h = lax.dot_general(
        feat_ref[...], w_ref[...],
        dimension_numbers=(((1,), (1,)), ((), ())),
        preferred_element_type=jnp.float32,
    )
    out_ref[...] = h + b_ref[...]


def _linear(feat, W, b):
    rb = 1000
    return pl.pallas_call(
        _linear_body,
        grid=(N_NODES // rb,),
        in_specs=[
            pl.BlockSpec((rb, D), lambda i: (i, 0)),
            pl.BlockSpec((D, D), lambda i: (0, 0)),
            pl.BlockSpec((1, D), lambda i: (0, 0)),
        ],
        out_specs=pl.BlockSpec((rb, D), lambda i: (i, 0)),
        out_shape=jax.ShapeDtypeStruct((N_NODES, D), jnp.float32),
    )(feat, W, b.reshape(1, D))


def _mp_body(h, zeros, src3, dst3, out, srcA, dstA, srcB, dstB,
             rows, acc, semG, semS, semSA, semDA, semSB, semDB):
    c = lax.axis_index("c")
    s = lax.axis_index("s")
    tid = c * NS + s

    z0 = s * ZERO_PER_TILE
    pltpu.sync_copy(zeros.at[pl.ds(z0, ZERO_PER_TILE)],
                    acc.at[pl.ds(z0, ZERO_PER_TILE)])

    def load_idx(blk, sbuf, dbuf, ssem, dsem):
        pltpu.async_copy(src3.at[tid, pl.ds(blk * IBLK, IBLK)], sbuf, ssem)
        pltpu.async_copy(dst3.at[tid, pl.ds(blk * IBLK, IBLK)], dbuf, dsem)

    def wait_idx(sbuf, dbuf, ssem, dsem):
        pltpu.make_async_copy(src3.at[tid, pl.ds(0, IBLK)], sbuf, ssem).wait()
        pltpu.make_async_copy(dst3.at[tid, pl.ds(0, IBLK)], dbuf, dsem).wait()

    def gather(idx_row, q):
        pltpu.async_copy(h.at[idx_row], rows[q], semG[q])

    def wait_gather(q):
        pltpu.make_async_copy(h.at[srcA.at[0]], rows[q], semG[q]).wait()

    def scatter(idx_row, q):
        pltpu.async_copy(rows[q], acc.at[idx_row], semS[q], add=True)

    def wait_scatter(q):
        pltpu.make_async_copy(rows[q], acc.at[dstA.at[0]], semS[q]).wait()

    def maybe_when(cond, fn):
        if isinstance(cond, bool):
            if cond:
                fn()
        else:
            pl.when(cond)(fn)

    load_idx(0, srcA, dstA, semSA, semDA)
    wait_idx(srcA, dstA, semSA, semDA)
    plsc.subcore_barrier()
    gather(srcA.at[0], 0)

    def half(blk, sbuf, dbuf, osbuf, odbuf, ossem, odsem, first=False):
        for j in range(IBLK):
            q = j % NRING
            qn = (j + 1) % NRING
            if not (first and j < NRING - 1):
                wait_scatter(qn)
            if j < IBLK - 1:
                gather(sbuf.at[j + 1], qn)
            else:
                def prime():
                    wait_idx(osbuf, odbuf, ossem, odsem)
                    gather(osbuf.at[0], qn)
                maybe_when(blk != BLOCKS - 1, prime)
            wait_gather(q)
            scatter(dbuf.at[j], q)
            if j == NRING - 2:
                maybe_when(blk + 1 < BLOCKS,
                           lambda: load_idx(blk + 1, osbuf, odbuf,
                                            ossem, odsem))

    half(0, srcA, dstA, srcB, dstB, semSB, semDB, first=True)
    half(1, srcB, dstB, srcA, dstA, semSA, semDA)

    @pl.loop(0, (BLOCKS - 2) // 2)
    def _(bp):
        blk = 2 + 2 * bp
        half(blk, srcA, dstA, srcB, dstB, semSB, semDB)
        half(blk + 1, srcB, dstB, srcA, dstA, semSA, semDA)

    for k in range(NRING - 1):
        wait_scatter((CPT - (NRING - 1) + k) % NRING)

    plsc.subcore_barrier()
    w0 = s * WRITE_PER_TILE
    pltpu.sync_copy(acc.at[pl.ds(w0, WRITE_PER_TILE)],
                    out.at[c, pl.ds(w0, WRITE_PER_TILE)])

    @pl.when(s == 0)
    def _():
        t0 = NS * WRITE_PER_TILE
        pltpu.sync_copy(acc.at[pl.ds(t0, WRITE_TAIL)],
                        out.at[c, pl.ds(t0, WRITE_TAIL)])


@functools.partial(
    pl.kernel,
    out_type=jax.ShapeDtypeStruct((NC, N_NODES, D), jnp.float32),
    mesh=plsc.VectorSubcoreMesh(core_axis_name="c", subcore_axis_name="s"),
    scratch_types=[
        pltpu.VMEM((IBLK, CHUNK), jnp.int32),
        pltpu.VMEM((IBLK, CHUNK), jnp.int32),
        pltpu.VMEM((IBLK, CHUNK), jnp.int32),
        pltpu.VMEM((IBLK, CHUNK), jnp.int32),
        [pltpu.VMEM((CHUNK, D), jnp.float32)] * NRING,
        pltpu.VMEM_SHARED((ACC_ROWS, D), jnp.float32),
        [pltpu.SemaphoreType.DMA] * NRING,
        [pltpu.SemaphoreType.DMA] * NRING,
        pltpu.SemaphoreType.DMA,
        pltpu.SemaphoreType.DMA,
        pltpu.SemaphoreType.DMA,
        pltpu.SemaphoreType.DMA,
    ],
)
def _message_passing(h, zeros, src3, dst3, out, srcA, dstA, srcB, dstB,
                     rows, acc, semG, semS, semSA, semDA, semSB, semDB):
    _mp_body(h, zeros, src3, dst3, out, srcA, dstA, srcB, dstB,
             rows, acc, semG, semS, semSA, semDA, semSB, semDB)


def _combine_body(p_ref, out_ref):
    out_ref[...] = p_ref[0] + p_ref[1]


def _combine(p):
    rb = 1000
    return pl.pallas_call(
        _combine_body,
        grid=(N_NODES // rb,),
        in_specs=[pl.BlockSpec((NC, rb, D), lambda i: (0, i, 0))],
        out_specs=pl.BlockSpec((rb, D), lambda i: (i, 0)),
        out_shape=jax.ShapeDtypeStruct((N_NODES, D), jnp.float32),
    )(p)


@jax.jit
def kernel(feat, edge_index, W, b):
    h = _linear(feat, W, b)
    npad = EDGES_PAD - N_EDGES
    src3 = jnp.concatenate(
        [edge_index[0], jnp.zeros((npad,), jnp.int32)]).reshape(NT, CPT, CHUNK)
    dst3 = jnp.concatenate(
        [edge_index[1], jnp.full((npad,), N_NODES, jnp.int32)]
    ).reshape(NT, CPT, CHUNK)
    zeros = jnp.zeros((ACC_ROWS, D), jnp.float32)
    return _combine(_message_passing(h, zeros, src3, dst3))
